# Initial kernel scaffold; baseline (speedup 1.0000x reference)
import jax
import jax.numpy as jnp
from jax.experimental import pallas as pl


def kernel(x, edge_index, edge_attr, edge_weights_index, pool_w, W0, w_ih, w_hh, b_ih, b_hh, lin1_w, lin1_b, lin2_w, lin2_b):
    raise NotImplementedError("write your pallas kernel here")


# speedup vs baseline: 8.1847x; 8.1847x over previous
import functools
import jax
import jax.numpy as jnp
from jax import lax
from jax.experimental import pallas as pl
from jax.experimental.pallas import tpu as pltpu
from jax.experimental.pallas import tpu_sc as plsc

N = 10000
D = 128
E = 320000

NC = 2
NS = 16
NW = NC * NS
PER = E // NW
CE = 80
NCHUNK = PER // CE
RPT = N // NS

f32 = jnp.float32
i32 = jnp.int32


def _score_body(x_ref, pw_ref, out_ref):
    pw = pw_ref[:]
    nrm = jnp.sqrt(jnp.sum(pw * pw))
    s = jnp.dot(x_ref[:], pw, preferred_element_type=f32) / nrm
    out_ref[:] = jnp.tanh(s)


def _score_call(x, pw2):
    return pl.pallas_call(
        _score_body,
        out_shape=jax.ShapeDtypeStruct((N, 1), f32),
    )(x, pw2)


def _evolve_body(s_ref, x_ref, w0_ref, wih_ref, whh_ref, bih_ref, bhh_ref,
                 xw_ref, xt_ref):
    R, C = s_ref.shape
    iota_lin = (lax.broadcasted_iota(i32, (R, C), 0) * C
                + lax.broadcasted_iota(i32, (R, C), 1))

    def body(i, s):
        m = jnp.max(s)
        lin = jnp.min(jnp.where(s == m, iota_lin, N))
        row = x_ref[pl.ds(lin, 1), :]
        xt_ref[pl.ds(i, 1), :] = row * m
        return jnp.where(iota_lin == lin, -2.0, s)

    lax.fori_loop(0, D, body, s_ref[:])

    xt = xt_ref[:]
    w0 = w0_ref[:]
    dn = (((1,), (1,)), ((), ()))
    gi = lax.dot_general(xt, wih_ref[:], dn, preferred_element_type=f32) + bih_ref[:]
    gh = lax.dot_general(w0, whh_ref[:], dn, preferred_element_type=f32) + bhh_ref[:]
    i_r, i_z, i_n = gi[:, :D], gi[:, D:2 * D], gi[:, 2 * D:]
    h_r, h_z, h_n = gh[:, :D], gh[:, D:2 * D], gh[:, 2 * D:]
    r = jax.nn.sigmoid(i_r + h_r)
    z = jax.nn.sigmoid(i_z + h_z)
    ncand = jnp.tanh(i_n + r * h_n)
    w = (1.0 - z) * ncand + z * w0
    xw_ref[:] = jnp.dot(x_ref[:], w, preferred_element_type=f32)


def _evolve_call(score2d, x, w0, w_ih, w_hh, b_ih2, b_hh2):
    return pl.pallas_call(
        _evolve_body,
        out_shape=jax.ShapeDtypeStruct((N, D), f32),
        scratch_shapes=[pltpu.VMEM((D, D), f32)],
    )(score2d, x, w0, w_ih, w_hh, b_ih2, b_hh2)


def _deg_body(col_hbm, attr_hbm, degp_hbm, colv, attrv, deg, sem):
    wid = lax.axis_index("s") * NC + lax.axis_index("c")
    base = wid * PER

    @pl.loop(0, N // 16, unroll=8)
    def _(j):
        deg[pl.ds(j * 16, 16)] = jnp.zeros((16,), f32)

    pltpu.sync_copy(col_hbm.at[pl.ds(base, PER)], colv)
    pltpu.sync_copy(attr_hbm.at[pl.ds(base, PER)], attrv)

    @pl.loop(0, PER // 16, unroll=8)
    def _(j):
        idx = colv[pl.ds(j * 16, 16)]
        a = attrv[pl.ds(j * 16, 16)]
        plsc.addupdate_scatter(deg, [idx], a)

    pltpu.sync_copy(deg, degp_hbm.at[wid])


def _deg_call(col, attr):
    mesh = plsc.VectorSubcoreMesh(core_axis_name="c", subcore_axis_name="s", num_cores=NC, num_subcores=NS)
    return pl.kernel(
        _deg_body,
        out_type=jax.ShapeDtypeStruct((NW, N), f32),
        mesh=mesh,
        compiler_params=pltpu.CompilerParams(
            needs_layout_passes=False, use_tc_tiling_on_sc=False),
        scratch_types=[
            pltpu.VMEM((PER,), i32),
            pltpu.VMEM((PER,), f32),
            pltpu.VMEM((N,), f32),
            pltpu.SemaphoreType.DMA,
        ],
    )(col, attr)


def _dis_body(degp_ref, xw_ref, xws_ref, dis_ref):
    deg = jnp.sum(degp_ref[:], axis=0)
    dis = jnp.where(deg > 0, lax.rsqrt(jnp.maximum(deg, 1e-12)), 0.0)
    disc = dis[:, None]
    dis_ref[:] = disc
    xws_ref[:] = xw_ref[:] * disc


def _dis_call(degp, xw):
    return pl.pallas_call(
        _dis_body,
        out_shape=[jax.ShapeDtypeStruct((N, D), f32),
                   jax.ShapeDtypeStruct((N, 1), f32)],
    )(degp, xw)


def _mp_body(xws_hbm, row_hbm, col_hbm, attr_hbm, zrows_hbm, p_hbm,
             rowv0, rowv1, colv0, colv1, attrv0, attrv1, rows0, rows1,
             acc, gsem0, gsem1, ssem0, ssem1):
    rowv = [rowv0, rowv1]
    colv = [colv0, colv1]
    attrv = [attrv0, attrv1]
    rows = [rows0, rows1]
    gsem = [gsem0, gsem1]
    ssem = [ssem0, ssem1]
    c = lax.axis_index("c")
    s = lax.axis_index("s")
    wid = s * NC + c
    base = wid * PER

    pltpu.sync_copy(zrows_hbm, acc.at[pl.ds(s * RPT, RPT)])
    plsc.subcore_barrier()

    def scale(rows, attrv):
        for g in range(CE // 16):
            a16 = attrv[pl.ds(g * 16, 16)]
            for e16 in range(16):
                e = g * 16 + e16
                av = a16.at[jnp.full((16,), e16, i32)].get(
                    mode="promise_in_bounds")
                for j in range(D // 16):
                    sl = pl.ds(j * 16, 16)
                    rows[e, sl] = rows[e, sl] * av

    def stage(off, p):
        pltpu.sync_copy(row_hbm.at[pl.ds(off, CE)], rowv[p])
        pltpu.sync_copy(col_hbm.at[pl.ds(off, CE)], colv[p])
        pltpu.sync_copy(attr_hbm.at[pl.ds(off, CE)], attrv[p])
        return pltpu.async_copy(xws_hbm.at[rowv[p]], rows[p], gsem[p])

    @pl.loop(0, NCHUNK // 2)
    def _(t):
        off0 = base + (2 * t) * CE
        g0 = stage(off0, 0)
        g1 = stage(off0 + CE, 1)
        g0.wait()
        scale(rows[0], attrv[0])
        s0 = pltpu.async_copy(rows[0], acc.at[colv[0]], ssem[0], add=True)
        g1.wait()
        scale(rows[1], attrv[1])
        s1 = pltpu.async_copy(rows[1], acc.at[colv[1]], ssem[1], add=True)
        s0.wait()
        s1.wait()

    if NCHUNK % 2:
        gt = stage(base + (NCHUNK - 1) * CE, 0)
        gt.wait()
        scale(rows[0], attrv[0])
        pltpu.async_copy(rows[0], acc.at[colv[0]], ssem[0], add=True).wait()

    plsc.subcore_barrier()
    pltpu.sync_copy(acc.at[pl.ds(s * RPT, RPT)], p_hbm.at[c, pl.ds(s * RPT, RPT)])


def _mp_call(xws, row, col, attr, zrows):
    mesh = plsc.VectorSubcoreMesh(core_axis_name="c", subcore_axis_name="s", num_cores=NC, num_subcores=NS)
    return pl.kernel(
        _mp_body,
        out_type=jax.ShapeDtypeStruct((NC, N, D), f32),
        mesh=mesh,
        compiler_params=pltpu.CompilerParams(
            needs_layout_passes=False, use_tc_tiling_on_sc=False),
        scratch_types=[
            pltpu.VMEM((CE,), i32),
            pltpu.VMEM((CE,), i32),
            pltpu.VMEM((CE,), i32),
            pltpu.VMEM((CE,), i32),
            pltpu.VMEM((CE,), f32),
            pltpu.VMEM((CE,), f32),
            pltpu.VMEM((CE, D), f32),
            pltpu.VMEM((CE, D), f32),
            pltpu.VMEM_SHARED((N, D), f32),
            pltpu.SemaphoreType.DMA,
            pltpu.SemaphoreType.DMA,
            pltpu.SemaphoreType.DMA,
            pltpu.SemaphoreType.DMA,
        ],
    )(xws, row, col, attr, zrows)


def _ab_body(p0_ref, p1_ref, dis_ref, l1_ref, b1_ref, a_ref, b_ref):
    ne = (p0_ref[:] + p1_ref[:]) * dis_ref[:]
    l1 = l1_ref[:]
    dn = (((1,), (1,)), ((), ()))
    a_ref[:] = lax.dot_general(ne, l1[:, :D], dn, preferred_element_type=f32) + b1_ref[:]
    b_ref[:] = lax.dot_general(ne, l1[:, D:], dn, preferred_element_type=f32)


def _ab_call(p0, p1, dis, lin1_w, b1):
    return pl.pallas_call(
        _ab_body,
        out_shape=[jax.ShapeDtypeStruct((N, D), f32),
                   jax.ShapeDtypeStruct((N, D), f32)],
    )(p0, p1, dis, lin1_w, b1)


def _dec_body(a_hbm, b_hbm, src_hbm, trg_hbm, w2_hbm, pred_hbm,
              srcv0, srcv1, trgv0, trgv1, rowsa0, rowsa1, rowsb0, rowsb1,
              w2v, outv0, outv1, sa0, sa1, sb0, sb1):
    srcv = [srcv0, srcv1]
    trgv = [trgv0, trgv1]
    rowsa = [rowsa0, rowsa1]
    rowsb = [rowsb0, rowsb1]
    outv = [outv0, outv1]
    sa = [sa0, sa1]
    sb = [sb0, sb1]
    wid = lax.axis_index("s") * NC + lax.axis_index("c")
    base = wid * PER

    pltpu.sync_copy(w2_hbm, w2v)
    wv = [w2v[pl.ds(j * 16, 16)] for j in range(D // 16)]
    lane = lax.iota(i32, 16)

    def stage(off, p):
        pltpu.sync_copy(src_hbm.at[pl.ds(off, CE)], srcv[p])
        pltpu.sync_copy(trg_hbm.at[pl.ds(off, CE)], trgv[p])
        return (pltpu.async_copy(a_hbm.at[srcv[p]], rowsa[p], sa[p]),
                pltpu.async_copy(b_hbm.at[trgv[p]], rowsb[p], sb[p]))

    def compute(off, p):
        for g in range(CE // 16):
            tv = jnp.zeros((16,), f32)
            for e16 in range(16):
                e = g * 16 + e16
                acc = jnp.zeros((16,), f32)
                for j in range(D // 16):
                    sl = pl.ds(j * 16, 16)
                    h = jnp.maximum(rowsa[p][e, sl] + rowsb[p][e, sl], 0.0)
                    acc = acc + h * wv[j]
                tot = jnp.sum(acc)
                tv = jnp.where(lane == e16, tot, tv)
            outv[p][pl.ds(g * 16, 16)] = tv
        pltpu.sync_copy(outv[p], pred_hbm.at[pl.ds(off, CE)])

    @pl.loop(0, NCHUNK // 2)
    def _(t):
        off0 = base + (2 * t) * CE
        da0, db0 = stage(off0, 0)
        da1, db1 = stage(off0 + CE, 1)
        da0.wait()
        db0.wait()
        compute(off0, 0)
        da1.wait()
        db1.wait()
        compute(off0 + CE, 1)

    if NCHUNK % 2:
        offt = base + (NCHUNK - 1) * CE
        da0, db0 = stage(offt, 0)
        da0.wait()
        db0.wait()
        compute(offt, 0)


def _dec_call(a, b, src, trg, w2):
    mesh = plsc.VectorSubcoreMesh(core_axis_name="c", subcore_axis_name="s", num_cores=NC, num_subcores=NS)
    return pl.kernel(
        _dec_body,
        out_type=jax.ShapeDtypeStruct((E,), f32),
        mesh=mesh,
        compiler_params=pltpu.CompilerParams(
            needs_layout_passes=False, use_tc_tiling_on_sc=False),
        scratch_types=[
            pltpu.VMEM((CE,), i32),
            pltpu.VMEM((CE,), i32),
            pltpu.VMEM((CE,), i32),
            pltpu.VMEM((CE,), i32),
            pltpu.VMEM((CE, D), f32),
            pltpu.VMEM((CE, D), f32),
            pltpu.VMEM((CE, D), f32),
            pltpu.VMEM((CE, D), f32),
            pltpu.VMEM((D,), f32),
            pltpu.VMEM((CE,), f32),
            pltpu.VMEM((CE,), f32),
            pltpu.SemaphoreType.DMA,
            pltpu.SemaphoreType.DMA,
            pltpu.SemaphoreType.DMA,
            pltpu.SemaphoreType.DMA,
        ],
    )(a, b, src, trg, w2)


@jax.jit
def kernel(x, edge_index, edge_attr, edge_weights_index, pool_w, W0,
           w_ih, w_hh, b_ih, b_hh, lin1_w, lin1_b, lin2_w, lin2_b):
    row = edge_index[0]
    col = edge_index[1]
    src = edge_weights_index[0]
    trg = edge_weights_index[1]

    score = _score_call(x, pool_w.reshape(D, 1))
    score2d = score.reshape(125, 80)
    xw = _evolve_call(score2d, x, W0, w_ih, w_hh,
                      b_ih.reshape(1, 3 * D), b_hh.reshape(1, 3 * D))

    degp = _deg_call(col, edge_attr)
    xws, dis = _dis_call(degp, xw)

    zrows = jnp.zeros((RPT, D), f32)
    p = _mp_call(xws, row, col, edge_attr, zrows)

    a, b = _ab_call(p[0], p[1], dis, lin1_w, lin1_b.reshape(1, D))

    pred = _dec_call(a, b, src, trg, lin2_w.reshape(D))
    return pred + lin2_b[0]


# speedup vs baseline: 9.1089x; 1.1129x over previous
import functools
import jax
import jax.numpy as jnp
from jax import lax
from jax.experimental import pallas as pl
from jax.experimental.pallas import tpu as pltpu
from jax.experimental.pallas import tpu_sc as plsc

N = 10000
D = 128
E = 320000

NC = 2
NS = 16
NW = NC * NS
PER = E // NW
CE = 80
NCHUNK = PER // CE
RPT = N // NS

f32 = jnp.float32
i32 = jnp.int32


def _score_body(x_ref, pw_ref, out_ref):
    pw = pw_ref[:]
    nrm = jnp.sqrt(jnp.sum(pw * pw))
    s = jnp.dot(x_ref[:], pw, preferred_element_type=f32) / nrm
    out_ref[:] = jnp.tanh(s)


def _score_call(x, pw2):
    return pl.pallas_call(
        _score_body,
        out_shape=jax.ShapeDtypeStruct((N, 1), f32),
    )(x, pw2)


def _evolve_body(s_ref, x_ref, w0_ref, wih_ref, whh_ref, bih_ref, bhh_ref,
                 xw_ref, xt_ref):
    R, C = s_ref.shape
    iota_lin = (lax.broadcasted_iota(i32, (R, C), 0) * C
                + lax.broadcasted_iota(i32, (R, C), 1))

    def body(i, s):
        m = jnp.max(s)
        lin = jnp.min(jnp.where(s == m, iota_lin, N))
        row = x_ref[pl.ds(lin, 1), :]
        xt_ref[pl.ds(i, 1), :] = row * m
        return jnp.where(iota_lin == lin, -2.0, s)

    lax.fori_loop(0, D, body, s_ref[:])

    xt = xt_ref[:]
    w0 = w0_ref[:]
    dn = (((1,), (1,)), ((), ()))
    gi = lax.dot_general(xt, wih_ref[:], dn, preferred_element_type=f32) + bih_ref[:]
    gh = lax.dot_general(w0, whh_ref[:], dn, preferred_element_type=f32) + bhh_ref[:]
    i_r, i_z, i_n = gi[:, :D], gi[:, D:2 * D], gi[:, 2 * D:]
    h_r, h_z, h_n = gh[:, :D], gh[:, D:2 * D], gh[:, 2 * D:]
    r = jax.nn.sigmoid(i_r + h_r)
    z = jax.nn.sigmoid(i_z + h_z)
    ncand = jnp.tanh(i_n + r * h_n)
    w = (1.0 - z) * ncand + z * w0
    xw_ref[:] = jnp.dot(x_ref[:], w, preferred_element_type=f32)


def _evolve_call(score2d, x, w0, w_ih, w_hh, b_ih2, b_hh2):
    return pl.pallas_call(
        _evolve_body,
        out_shape=jax.ShapeDtypeStruct((N, D), f32),
        scratch_shapes=[pltpu.VMEM((D, D), f32)],
    )(score2d, x, w0, w_ih, w_hh, b_ih2, b_hh2)


def _deg_body(col_hbm, attr_hbm, degp_hbm, colv, attrv, deg, sem):
    wid = lax.axis_index("s") * NC + lax.axis_index("c")
    base = wid * PER

    @pl.loop(0, N // 16, unroll=8)
    def _(j):
        deg[pl.ds(j * 16, 16)] = jnp.zeros((16,), f32)

    pltpu.sync_copy(col_hbm.at[pl.ds(base, PER)], colv)
    pltpu.sync_copy(attr_hbm.at[pl.ds(base, PER)], attrv)

    @pl.loop(0, PER // 16, unroll=8)
    def _(j):
        idx = colv[pl.ds(j * 16, 16)]
        a = attrv[pl.ds(j * 16, 16)]
        plsc.addupdate_scatter(deg, [idx], a)

    pltpu.sync_copy(deg, degp_hbm.at[wid])


def _deg_call(col, attr):
    mesh = plsc.VectorSubcoreMesh(core_axis_name="c", subcore_axis_name="s", num_cores=NC, num_subcores=NS)
    return pl.kernel(
        _deg_body,
        out_type=jax.ShapeDtypeStruct((NW, N), f32),
        mesh=mesh,
        compiler_params=pltpu.CompilerParams(
            needs_layout_passes=False, use_tc_tiling_on_sc=False),
        scratch_types=[
            pltpu.VMEM((PER,), i32),
            pltpu.VMEM((PER,), f32),
            pltpu.VMEM((N,), f32),
            pltpu.SemaphoreType.DMA,
        ],
    )(col, attr)


def _dis_body(degp_ref, xw_ref, xws_ref, dis_ref):
    deg = jnp.sum(degp_ref[:], axis=0)
    dis = jnp.where(deg > 0, lax.rsqrt(jnp.maximum(deg, 1e-12)), 0.0)
    disc = dis[:, None]
    dis_ref[:] = disc
    xws_ref[:] = xw_ref[:] * disc


def _dis_call(degp, xw):
    return pl.pallas_call(
        _dis_body,
        out_shape=[jax.ShapeDtypeStruct((N, D), f32),
                   jax.ShapeDtypeStruct((N, 1), f32)],
    )(degp, xw)


def _mp_body(xws_hbm, row_hbm, col3_hbm, attr_hbm, zrows_hbm, p_hbm,
             rowf, colf2, attrf, rows0, rows1,
             acc, gsem0, gsem1, ssem0, ssem1):
    rows = [rows0, rows1]
    gsem = [gsem0, gsem1]
    ssem = [ssem0, ssem1]
    c = lax.axis_index("c")
    s = lax.axis_index("s")
    wid = s * NC + c
    base = wid * PER

    pltpu.sync_copy(zrows_hbm, acc.at[pl.ds(s * RPT, RPT)])
    pltpu.sync_copy(row_hbm.at[pl.ds(base, PER)], rowf)
    pltpu.sync_copy(col3_hbm.at[wid], colf2)
    pltpu.sync_copy(attr_hbm.at[pl.ds(base, PER)], attrf)
    plsc.subcore_barrier()

    def scale(p, ebase):
        for g in range(CE // 16):
            a16 = attrf[pl.ds(ebase + g * 16, 16)]
            for e16 in range(16):
                e = g * 16 + e16
                av = a16.at[jnp.full((16,), e16, i32)].get(
                    mode="promise_in_bounds")
                for j in range(D // 16):
                    sl = pl.ds(j * 16, 16)
                    rows[p][e, sl] = rows[p][e, sl] * av

    def stage(eoff, p):
        return pltpu.async_copy(
            xws_hbm.at[rowf.at[pl.ds(eoff, CE)]], rows[p], gsem[p])

    @pl.loop(0, NCHUNK // 2)
    def _(t):
        e0 = (2 * t) * CE
        g0 = stage(e0, 0)
        g1 = stage(e0 + CE, 1)
        g0.wait()
        scale(0, e0)
        s0 = pltpu.async_copy(rows[0], acc.at[colf2.at[2 * t]], ssem[0],
                              add=True)
        g1.wait()
        scale(1, e0 + CE)
        s1 = pltpu.async_copy(rows[1], acc.at[colf2.at[2 * t + 1]], ssem[1],
                              add=True)
        s0.wait()
        s1.wait()

    if NCHUNK % 2:
        et = (NCHUNK - 1) * CE
        gt = stage(et, 0)
        gt.wait()
        scale(0, et)
        pltpu.async_copy(rows[0], acc.at[colf2.at[NCHUNK - 1]], ssem[0],
                         add=True).wait()

    plsc.subcore_barrier()
    pltpu.sync_copy(acc.at[pl.ds(s * RPT, RPT)], p_hbm.at[c, pl.ds(s * RPT, RPT)])


def _mp_call(xws, row, col, attr, zrows):
    mesh = plsc.VectorSubcoreMesh(core_axis_name="c", subcore_axis_name="s", num_cores=NC, num_subcores=NS)
    return pl.kernel(
        _mp_body,
        out_type=jax.ShapeDtypeStruct((NC, N, D), f32),
        mesh=mesh,
        compiler_params=pltpu.CompilerParams(
            needs_layout_passes=False, use_tc_tiling_on_sc=False),
        scratch_types=[
            pltpu.VMEM((PER,), i32),
            pltpu.VMEM((NCHUNK, CE), i32),
            pltpu.VMEM((PER,), f32),
            pltpu.VMEM((CE, D), f32),
            pltpu.VMEM((CE, D), f32),
            pltpu.VMEM_SHARED((N, D), f32),
            pltpu.SemaphoreType.DMA,
            pltpu.SemaphoreType.DMA,
            pltpu.SemaphoreType.DMA,
            pltpu.SemaphoreType.DMA,
        ],
    )(xws, row, col.reshape(NW, NCHUNK, CE), attr, zrows)


def _ab_body(p0_ref, p1_ref, dis_ref, l1_ref, b1_ref, a_ref, b_ref):
    ne = (p0_ref[:] + p1_ref[:]) * dis_ref[:]
    l1 = l1_ref[:]
    dn = (((1,), (1,)), ((), ()))
    a_ref[:] = lax.dot_general(ne, l1[:, :D], dn, preferred_element_type=f32) + b1_ref[:]
    b_ref[:] = lax.dot_general(ne, l1[:, D:], dn, preferred_element_type=f32)


def _ab_call(p0, p1, dis, lin1_w, b1):
    return pl.pallas_call(
        _ab_body,
        out_shape=[jax.ShapeDtypeStruct((N, D), f32),
                   jax.ShapeDtypeStruct((N, D), f32)],
    )(p0, p1, dis, lin1_w, b1)


def _dec_body(a_hbm, b_hbm, src_hbm, trg_hbm, w2_hbm, pred_hbm,
              srcf, trgf, rowsa0, rowsa1, rowsb0, rowsb1,
              w2v, outv0, outv1, sa0, sa1, sb0, sb1):
    rowsa = [rowsa0, rowsa1]
    rowsb = [rowsb0, rowsb1]
    outv = [outv0, outv1]
    sa = [sa0, sa1]
    sb = [sb0, sb1]
    wid = lax.axis_index("s") * NC + lax.axis_index("c")
    base = wid * PER

    pltpu.sync_copy(w2_hbm, w2v)
    pltpu.sync_copy(src_hbm.at[pl.ds(base, PER)], srcf)
    pltpu.sync_copy(trg_hbm.at[pl.ds(base, PER)], trgf)
    wv = [w2v[pl.ds(j * 16, 16)] for j in range(D // 16)]
    lane = lax.iota(i32, 16)
    bperm = [jnp.bitwise_xor(lane, k) for k in (8, 4, 2, 1)]

    def stage(eoff, p):
        return (pltpu.async_copy(a_hbm.at[srcf.at[pl.ds(eoff, CE)]],
                                 rowsa[p], sa[p]),
                pltpu.async_copy(b_hbm.at[trgf.at[pl.ds(eoff, CE)]],
                                 rowsb[p], sb[p]))

    def compute(eoff, p):
        for g in range(CE // 16):
            tv = jnp.zeros((16,), f32)
            for e16 in range(16):
                e = g * 16 + e16
                acc = jnp.zeros((16,), f32)
                for j in range(D // 16):
                    sl = pl.ds(j * 16, 16)
                    h = jnp.maximum(rowsa[p][e, sl] + rowsb[p][e, sl], 0.0)
                    acc = acc + h * wv[j]
                for pm in bperm:
                    acc = acc + acc.at[pm].get(mode="promise_in_bounds")
                tv = jnp.where(lane == e16, acc, tv)
            outv[p][pl.ds(g * 16, 16)] = tv
        pltpu.sync_copy(outv[p], pred_hbm.at[pl.ds(base + eoff, CE)])

    @pl.loop(0, NCHUNK // 2)
    def _(t):
        e0 = (2 * t) * CE
        da0, db0 = stage(e0, 0)
        da1, db1 = stage(e0 + CE, 1)
        da0.wait()
        db0.wait()
        compute(e0, 0)
        da1.wait()
        db1.wait()
        compute(e0 + CE, 1)

    if NCHUNK % 2:
        et = (NCHUNK - 1) * CE
        da0, db0 = stage(et, 0)
        da0.wait()
        db0.wait()
        compute(et, 0)


def _dec_call(a, b, src, trg, w2):
    mesh = plsc.VectorSubcoreMesh(core_axis_name="c", subcore_axis_name="s", num_cores=NC, num_subcores=NS)
    return pl.kernel(
        _dec_body,
        out_type=jax.ShapeDtypeStruct((E,), f32),
        mesh=mesh,
        compiler_params=pltpu.CompilerParams(
            needs_layout_passes=False, use_tc_tiling_on_sc=False),
        scratch_types=[
            pltpu.VMEM((PER,), i32),
            pltpu.VMEM((PER,), i32),
            pltpu.VMEM((CE, D), f32),
            pltpu.VMEM((CE, D), f32),
            pltpu.VMEM((CE, D), f32),
            pltpu.VMEM((CE, D), f32),
            pltpu.VMEM((D,), f32),
            pltpu.VMEM((CE,), f32),
            pltpu.VMEM((CE,), f32),
            pltpu.SemaphoreType.DMA,
            pltpu.SemaphoreType.DMA,
            pltpu.SemaphoreType.DMA,
            pltpu.SemaphoreType.DMA,
        ],
    )(a, b, src, trg, w2)


@jax.jit
def kernel(x, edge_index, edge_attr, edge_weights_index, pool_w, W0,
           w_ih, w_hh, b_ih, b_hh, lin1_w, lin1_b, lin2_w, lin2_b):
    row = edge_index[0]
    col = edge_index[1]
    src = edge_weights_index[0]
    trg = edge_weights_index[1]

    score = _score_call(x, pool_w.reshape(D, 1))
    score2d = score.reshape(125, 80)
    xw = _evolve_call(score2d, x, W0, w_ih, w_hh,
                      b_ih.reshape(1, 3 * D), b_hh.reshape(1, 3 * D))

    degp = _deg_call(col, edge_attr)
    xws, dis = _dis_call(degp, xw)

    zrows = jnp.zeros((RPT, D), f32)
    p = _mp_call(xws, row, col, edge_attr, zrows)

    a, b = _ab_call(p[0], p[1], dis, lin1_w, lin1_b.reshape(1, D))

    pred = _dec_call(a, b, src, trg, lin2_w.reshape(D))
    return pred + lin2_b[0]


# speedup vs baseline: 11.2875x; 1.2392x over previous
import functools
import jax
import jax.numpy as jnp
from jax import lax
from jax.experimental import pallas as pl
from jax.experimental.pallas import tpu as pltpu
from jax.experimental.pallas import tpu_sc as plsc

N = 10000
D = 128
E = 320000

NC = 2
NS = 16
NW = NC * NS
PER = E // NW
CE = 80
NCHUNK = PER // CE
RPT = N // NS

f32 = jnp.float32
i32 = jnp.int32


def _score_body(x_ref, pw_ref, out_ref):
    pw = pw_ref[:]
    nrm = jnp.sqrt(jnp.sum(pw * pw))
    s = jnp.dot(x_ref[:], pw, preferred_element_type=f32) / nrm
    out_ref[:] = jnp.tanh(s)


def _score_call(x, pw2):
    return pl.pallas_call(
        _score_body,
        out_shape=jax.ShapeDtypeStruct((N, 1), f32),
    )(x, pw2)


def _evolve_body(s_ref, x_ref, w0_ref, wih_ref, whh_ref, bih_ref, bhh_ref,
                 xw_ref, xt_ref):
    R, C = s_ref.shape
    iota_lin = (lax.broadcasted_iota(i32, (R, C), 0) * C
                + lax.broadcasted_iota(i32, (R, C), 1))

    def body(i, s):
        m = jnp.max(s)
        lin = jnp.min(jnp.where(s == m, iota_lin, N))
        row = x_ref[pl.ds(lin, 1), :]
        xt_ref[pl.ds(i, 1), :] = row * m
        return jnp.where(iota_lin == lin, -2.0, s)

    lax.fori_loop(0, D, body, s_ref[:])

    xt = xt_ref[:]
    w0 = w0_ref[:]
    dn = (((1,), (1,)), ((), ()))
    gi = lax.dot_general(xt, wih_ref[:], dn, preferred_element_type=f32) + bih_ref[:]
    gh = lax.dot_general(w0, whh_ref[:], dn, preferred_element_type=f32) + bhh_ref[:]
    i_r, i_z, i_n = gi[:, :D], gi[:, D:2 * D], gi[:, 2 * D:]
    h_r, h_z, h_n = gh[:, :D], gh[:, D:2 * D], gh[:, 2 * D:]
    r = jax.nn.sigmoid(i_r + h_r)
    z = jax.nn.sigmoid(i_z + h_z)
    ncand = jnp.tanh(i_n + r * h_n)
    w = (1.0 - z) * ncand + z * w0
    xw_ref[:] = jnp.dot(x_ref[:], w, preferred_element_type=f32)


def _evolve_call(score2d, x, w0, w_ih, w_hh, b_ih2, b_hh2):
    return pl.pallas_call(
        _evolve_body,
        out_shape=jax.ShapeDtypeStruct((N, D), f32),
        scratch_shapes=[pltpu.VMEM((D, D), f32)],
    )(score2d, x, w0, w_ih, w_hh, b_ih2, b_hh2)


def _deg_body(col_hbm, attr_hbm, degp_hbm, colv, attrv, deg, sem):
    wid = lax.axis_index("s") * NC + lax.axis_index("c")
    base = wid * PER

    @pl.loop(0, N // 16, unroll=8)
    def _(j):
        deg[pl.ds(j * 16, 16)] = jnp.zeros((16,), f32)

    pltpu.sync_copy(col_hbm.at[pl.ds(base, PER)], colv)
    pltpu.sync_copy(attr_hbm.at[pl.ds(base, PER)], attrv)

    @pl.loop(0, PER // 16, unroll=8)
    def _(j):
        idx = colv[pl.ds(j * 16, 16)]
        a = attrv[pl.ds(j * 16, 16)]
        plsc.addupdate_scatter(deg, [idx], a)

    pltpu.sync_copy(deg, degp_hbm.at[wid])


def _deg_call(col, attr):
    mesh = plsc.VectorSubcoreMesh(core_axis_name="c", subcore_axis_name="s", num_cores=NC, num_subcores=NS)
    return pl.kernel(
        _deg_body,
        out_type=jax.ShapeDtypeStruct((NW, N), f32),
        mesh=mesh,
        compiler_params=pltpu.CompilerParams(
            needs_layout_passes=False, use_tc_tiling_on_sc=False),
        scratch_types=[
            pltpu.VMEM((PER,), i32),
            pltpu.VMEM((PER,), f32),
            pltpu.VMEM((N,), f32),
            pltpu.SemaphoreType.DMA,
        ],
    )(col, attr)


def _dis_body(degp_ref, xw_ref, xws_ref, dis_ref):
    deg = jnp.sum(degp_ref[:], axis=0)
    dis = jnp.where(deg > 0, lax.rsqrt(jnp.maximum(deg, 1e-12)), 0.0)
    disc = dis[:, None]
    dis_ref[:] = disc
    xws_ref[:] = xw_ref[:] * disc


def _dis_call(degp, xw):
    return pl.pallas_call(
        _dis_body,
        out_shape=[jax.ShapeDtypeStruct((N, D), f32),
                   jax.ShapeDtypeStruct((N, 1), f32)],
    )(degp, xw)


def _mp_body(xws_hbm, row_hbm, col3_hbm, attr_hbm, zrows_hbm, p_hbm,
             rowf, colf2, attrf, rows0, rows1,
             acc, gsem0, gsem1, ssem0, ssem1):
    rows = [rows0, rows1]
    gsem = [gsem0, gsem1]
    ssem = [ssem0, ssem1]
    c = lax.axis_index("c")
    s = lax.axis_index("s")
    wid = s * NC + c
    base = wid * PER

    pltpu.sync_copy(zrows_hbm, acc.at[pl.ds(s * RPT, RPT)])
    pltpu.sync_copy(row_hbm.at[pl.ds(base, PER)], rowf)
    pltpu.sync_copy(col3_hbm.at[wid], colf2)
    pltpu.sync_copy(attr_hbm.at[pl.ds(base, PER)], attrf)
    plsc.subcore_barrier()

    def scale(p, ebase):
        for g in range(CE // 16):
            a16 = attrf[pl.ds(ebase + g * 16, 16)]
            for e16 in range(16):
                e = g * 16 + e16
                av = a16.at[jnp.full((16,), e16, i32)].get(
                    mode="promise_in_bounds")
                for j in range(D // 16):
                    sl = pl.ds(j * 16, 16)
                    rows[p][e, sl] = rows[p][e, sl] * av

    def stage(eoff, p):
        return pltpu.async_copy(
            xws_hbm.at[rowf.at[pl.ds(eoff, CE)]], rows[p], gsem[p])

    @pl.loop(0, NCHUNK // 2)
    def _(t):
        e0 = (2 * t) * CE
        g0 = stage(e0, 0)
        g1 = stage(e0 + CE, 1)
        g0.wait()
        scale(0, e0)
        s0 = pltpu.async_copy(rows[0], acc.at[colf2.at[2 * t]], ssem[0],
                              add=True)
        g1.wait()
        scale(1, e0 + CE)
        s1 = pltpu.async_copy(rows[1], acc.at[colf2.at[2 * t + 1]], ssem[1],
                              add=True)
        s0.wait()
        s1.wait()

    if NCHUNK % 2:
        et = (NCHUNK - 1) * CE
        gt = stage(et, 0)
        gt.wait()
        scale(0, et)
        pltpu.async_copy(rows[0], acc.at[colf2.at[NCHUNK - 1]], ssem[0],
                         add=True).wait()

    plsc.subcore_barrier()
    pltpu.sync_copy(acc.at[pl.ds(s * RPT, RPT)], p_hbm.at[c, pl.ds(s * RPT, RPT)])


def _mp_call(xws, row, col, attr, zrows):
    mesh = plsc.VectorSubcoreMesh(core_axis_name="c", subcore_axis_name="s", num_cores=NC, num_subcores=NS)
    return pl.kernel(
        _mp_body,
        out_type=jax.ShapeDtypeStruct((NC, N, D), f32),
        mesh=mesh,
        compiler_params=pltpu.CompilerParams(
            needs_layout_passes=False, use_tc_tiling_on_sc=False),
        scratch_types=[
            pltpu.VMEM((PER,), i32),
            pltpu.VMEM((NCHUNK, CE), i32),
            pltpu.VMEM((PER,), f32),
            pltpu.VMEM((CE, D), f32),
            pltpu.VMEM((CE, D), f32),
            pltpu.VMEM_SHARED((N, D), f32),
            pltpu.SemaphoreType.DMA,
            pltpu.SemaphoreType.DMA,
            pltpu.SemaphoreType.DMA,
            pltpu.SemaphoreType.DMA,
        ],
    )(xws, row, col.reshape(NW, NCHUNK, CE), attr, zrows)


def _ab_body(p0_ref, p1_ref, dis_ref, l1_ref, b1_ref, a_ref, b_ref):
    ne = (p0_ref[:] + p1_ref[:]) * dis_ref[:]
    l1 = l1_ref[:]
    dn = (((1,), (1,)), ((), ()))
    a_ref[:] = lax.dot_general(ne, l1[:, :D], dn, preferred_element_type=f32) + b1_ref[:]
    b_ref[:] = lax.dot_general(ne, l1[:, D:], dn, preferred_element_type=f32)


def _ab_call(p0, p1, dis, lin1_w, b1):
    return pl.pallas_call(
        _ab_body,
        out_shape=[jax.ShapeDtypeStruct((N, D), f32),
                   jax.ShapeDtypeStruct((N, D), f32)],
    )(p0, p1, dis, lin1_w, b1)


def _dec_body(a_hbm, b_hbm, src_hbm, trg_hbm, w2_hbm, pred_hbm,
              srcf, trgf, rowsa0, rowsa1, rowsb0, rowsb1,
              w2v, outv0, outv1, sa0, sa1, sb0, sb1):
    rowsa = [rowsa0, rowsa1]
    rowsb = [rowsb0, rowsb1]
    outv = [outv0, outv1]
    sa = [sa0, sa1]
    sb = [sb0, sb1]
    wid = lax.axis_index("s") * NC + lax.axis_index("c")
    base = wid * PER

    pltpu.sync_copy(w2_hbm, w2v)
    pltpu.sync_copy(src_hbm.at[pl.ds(base, PER)], srcf)
    pltpu.sync_copy(trg_hbm.at[pl.ds(base, PER)], trgf)
    wv = [w2v[pl.ds(j * 16, 16)] for j in range(D // 16)]
    lane = lax.iota(i32, 16)
    bperm = [jnp.bitwise_xor(lane, k) for k in (8, 4, 2, 1)]

    def stage(eoff, p):
        return (pltpu.async_copy(a_hbm.at[srcf.at[pl.ds(eoff, CE)]],
                                 rowsa[p], sa[p]),
                pltpu.async_copy(b_hbm.at[trgf.at[pl.ds(eoff, CE)]],
                                 rowsb[p], sb[p]))

    def compute(eoff, p):
        for g in range(CE // 16):
            tv = jnp.zeros((16,), f32)
            for e16 in range(16):
                e = g * 16 + e16
                acc = jnp.zeros((16,), f32)
                for j in range(D // 16):
                    sl = pl.ds(j * 16, 16)
                    h = jnp.maximum(rowsa[p][e, sl] + rowsb[p][e, sl], 0.0)
                    acc = acc + h * wv[j]
                for pm in bperm:
                    acc = acc + acc.at[pm].get(mode="promise_in_bounds")
                tv = jnp.where(lane == e16, acc, tv)
            outv[p][pl.ds(g * 16, 16)] = tv
        pltpu.sync_copy(outv[p], pred_hbm.at[pl.ds(base + eoff, CE)])

    def wait_set(p):
        pltpu.make_async_copy(a_hbm.at[srcf.at[pl.ds(0, CE)]],
                              rowsa[p], sa[p]).wait()
        pltpu.make_async_copy(b_hbm.at[trgf.at[pl.ds(0, CE)]],
                              rowsb[p], sb[p]).wait()

    stage(0, 0)
    stage(CE, 1)

    @pl.loop(0, NCHUNK // 2)
    def _(t):
        e0 = (2 * t) * CE
        wait_set(0)
        compute(e0, 0)
        stage(e0 + 2 * CE, 0)
        wait_set(1)
        compute(e0 + CE, 1)

        @pl.when(2 * t + 3 < NCHUNK)
        def _():
            stage(e0 + 3 * CE, 1)

    if NCHUNK % 2:
        wait_set(0)
        compute((NCHUNK - 1) * CE, 0)


def _dec_call(a, b, src, trg, w2):
    mesh = plsc.VectorSubcoreMesh(core_axis_name="c", subcore_axis_name="s", num_cores=NC, num_subcores=NS)
    return pl.kernel(
        _dec_body,
        out_type=jax.ShapeDtypeStruct((E,), f32),
        mesh=mesh,
        compiler_params=pltpu.CompilerParams(
            needs_layout_passes=False, use_tc_tiling_on_sc=False),
        scratch_types=[
            pltpu.VMEM((PER,), i32),
            pltpu.VMEM((PER,), i32),
            pltpu.VMEM((CE, D), f32),
            pltpu.VMEM((CE, D), f32),
            pltpu.VMEM((CE, D), f32),
            pltpu.VMEM((CE, D), f32),
            pltpu.VMEM((D,), f32),
            pltpu.VMEM((CE,), f32),
            pltpu.VMEM((CE,), f32),
            pltpu.SemaphoreType.DMA,
            pltpu.SemaphoreType.DMA,
            pltpu.SemaphoreType.DMA,
            pltpu.SemaphoreType.DMA,
        ],
    )(a, b, src, trg, w2)


@jax.jit
def kernel(x, edge_index, edge_attr, edge_weights_index, pool_w, W0,
           w_ih, w_hh, b_ih, b_hh, lin1_w, lin1_b, lin2_w, lin2_b):
    row = edge_index[0]
    col = edge_index[1]
    src = edge_weights_index[0]
    trg = edge_weights_index[1]

    score = _score_call(x, pool_w.reshape(D, 1))
    score2d = score.reshape(125, 80)
    xw = _evolve_call(score2d, x, W0, w_ih, w_hh,
                      b_ih.reshape(1, 3 * D), b_hh.reshape(1, 3 * D))

    degp = _deg_call(col, edge_attr)
    xws, dis = _dis_call(degp, xw)

    zrows = jnp.zeros((RPT, D), f32)
    p = _mp_call(xws, row, col, edge_attr, zrows)

    a, b = _ab_call(p[0], p[1], dis, lin1_w, lin1_b.reshape(1, D))

    pred = _dec_call(a, b, src, trg, lin2_w.reshape(D))
    return pred + lin2_b[0]


# speedup vs baseline: 12.2658x; 1.0867x over previous
import functools
import jax
import jax.numpy as jnp
from jax import lax
from jax.experimental import pallas as pl
from jax.experimental.pallas import tpu as pltpu
from jax.experimental.pallas import tpu_sc as plsc

N = 10000
D = 128
E = 320000

NC = 2
NS = 16
NW = NC * NS
PER = E // NW
CE = 80
NCHUNK = PER // CE
RPT = N // NS

f32 = jnp.float32
i32 = jnp.int32


def _score_body(x_ref, pw_ref, out_ref):
    pw = pw_ref[:]
    nrm = jnp.sqrt(jnp.sum(pw * pw))
    s = jnp.dot(x_ref[:], pw, preferred_element_type=f32) / nrm
    out_ref[:] = jnp.tanh(s)


def _score_call(x, pw2):
    return pl.pallas_call(
        _score_body,
        out_shape=jax.ShapeDtypeStruct((N, 1), f32),
    )(x, pw2)


def _evolve_body(s_ref, x_ref, w0_ref, wih_ref, whh_ref, bih_ref, bhh_ref,
                 xw_ref, xt_ref):
    R, C = s_ref.shape
    iota_lin = (lax.broadcasted_iota(i32, (R, C), 0) * C
                + lax.broadcasted_iota(i32, (R, C), 1))

    def body(i, s):
        m = jnp.max(s)
        lin = jnp.min(jnp.where(s == m, iota_lin, N))
        row = x_ref[pl.ds(lin, 1), :]
        xt_ref[pl.ds(i, 1), :] = row * m
        return jnp.where(iota_lin == lin, -2.0, s)

    lax.fori_loop(0, D, body, s_ref[:])

    xt = xt_ref[:]
    w0 = w0_ref[:]
    dn = (((1,), (1,)), ((), ()))
    gi = lax.dot_general(xt, wih_ref[:], dn, preferred_element_type=f32) + bih_ref[:]
    gh = lax.dot_general(w0, whh_ref[:], dn, preferred_element_type=f32) + bhh_ref[:]
    i_r, i_z, i_n = gi[:, :D], gi[:, D:2 * D], gi[:, 2 * D:]
    h_r, h_z, h_n = gh[:, :D], gh[:, D:2 * D], gh[:, 2 * D:]
    r = jax.nn.sigmoid(i_r + h_r)
    z = jax.nn.sigmoid(i_z + h_z)
    ncand = jnp.tanh(i_n + r * h_n)
    w = (1.0 - z) * ncand + z * w0
    xw_ref[:] = jnp.dot(x_ref[:], w, preferred_element_type=f32)


def _evolve_call(score2d, x, w0, w_ih, w_hh, b_ih2, b_hh2):
    return pl.pallas_call(
        _evolve_body,
        out_shape=jax.ShapeDtypeStruct((N, D), f32),
        scratch_shapes=[pltpu.VMEM((D, D), f32)],
    )(score2d, x, w0, w_ih, w_hh, b_ih2, b_hh2)


def _deg_body(col_hbm, attr_hbm, degp_hbm, colv, attrv, deg, sem):
    wid = lax.axis_index("s") * NC + lax.axis_index("c")
    base = wid * PER

    @pl.loop(0, N // 16, unroll=8)
    def _(j):
        deg[pl.ds(j * 16, 16)] = jnp.zeros((16,), f32)

    pltpu.sync_copy(col_hbm.at[pl.ds(base, PER)], colv)
    pltpu.sync_copy(attr_hbm.at[pl.ds(base, PER)], attrv)

    @pl.loop(0, PER // 16, unroll=8)
    def _(j):
        idx = colv[pl.ds(j * 16, 16)]
        a = attrv[pl.ds(j * 16, 16)]
        plsc.addupdate_scatter(deg, [idx], a)

    pltpu.sync_copy(deg, degp_hbm.at[wid])


def _deg_call(col, attr):
    mesh = plsc.VectorSubcoreMesh(core_axis_name="c", subcore_axis_name="s", num_cores=NC, num_subcores=NS)
    return pl.kernel(
        _deg_body,
        out_type=jax.ShapeDtypeStruct((NW, N), f32),
        mesh=mesh,
        compiler_params=pltpu.CompilerParams(
            needs_layout_passes=False, use_tc_tiling_on_sc=False),
        scratch_types=[
            pltpu.VMEM((PER,), i32),
            pltpu.VMEM((PER,), f32),
            pltpu.VMEM((N,), f32),
            pltpu.SemaphoreType.DMA,
        ],
    )(col, attr)


def _dis_body(degp_ref, xw_ref, xws_ref, dis_ref):
    deg = jnp.sum(degp_ref[:], axis=0)
    dis = jnp.where(deg > 0, lax.rsqrt(jnp.maximum(deg, 1e-12)), 0.0)
    disc = dis[:, None]
    dis_ref[:] = disc
    xws_ref[:] = xw_ref[:] * disc


def _dis_call(degp, xw):
    return pl.pallas_call(
        _dis_body,
        out_shape=[jax.ShapeDtypeStruct((N, D), f32),
                   jax.ShapeDtypeStruct((N, 1), f32)],
    )(degp, xw)


def _mp_body(xws_hbm, row_hbm, col3_hbm, attr_hbm, zrows_hbm, p_hbm,
             rowf, colf2, attrf, rows0, rows1,
             acc, gsem0, gsem1, ssem0, ssem1):
    rows = [rows0, rows1]
    gsem = [gsem0, gsem1]
    ssem = [ssem0, ssem1]
    c = lax.axis_index("c")
    s = lax.axis_index("s")
    wid = s * NC + c
    base = wid * PER

    pltpu.sync_copy(zrows_hbm, acc.at[pl.ds(s * RPT, RPT)])
    pltpu.sync_copy(row_hbm.at[pl.ds(base, PER)], rowf)
    pltpu.sync_copy(col3_hbm.at[wid], colf2)
    pltpu.sync_copy(attr_hbm.at[pl.ds(base, PER)], attrf)
    plsc.subcore_barrier()

    def scale(p, ebase):
        for g in range(CE // 16):
            a16 = attrf[pl.ds(ebase + g * 16, 16)]
            for e16 in range(16):
                e = g * 16 + e16
                av = a16.at[jnp.full((16,), e16, i32)].get(
                    mode="promise_in_bounds")
                for j in range(D // 16):
                    sl = pl.ds(j * 16, 16)
                    rows[p][e, sl] = rows[p][e, sl] * av

    def stage(eoff, p):
        return pltpu.async_copy(
            xws_hbm.at[rowf.at[pl.ds(eoff, CE)]], rows[p], gsem[p])

    def wait_gather(p):
        pltpu.make_async_copy(xws_hbm.at[rowf.at[pl.ds(0, CE)]],
                              rows[p], gsem[p]).wait()

    stage(0, 0)
    stage(CE, 1)

    @pl.loop(0, NCHUNK // 2)
    def _(t):
        e0 = (2 * t) * CE
        wait_gather(0)
        scale(0, e0)
        s0 = pltpu.async_copy(rows[0], acc.at[colf2.at[2 * t]], ssem[0],
                              add=True)
        wait_gather(1)
        scale(1, e0 + CE)
        s1 = pltpu.async_copy(rows[1], acc.at[colf2.at[2 * t + 1]], ssem[1],
                              add=True)
        s0.wait()
        stage(e0 + 2 * CE, 0)
        s1.wait()

        @pl.when(2 * t + 3 < NCHUNK)
        def _():
            stage(e0 + 3 * CE, 1)

    if NCHUNK % 2:
        wait_gather(0)
        scale(0, (NCHUNK - 1) * CE)
        pltpu.async_copy(rows[0], acc.at[colf2.at[NCHUNK - 1]], ssem[0],
                         add=True).wait()

    plsc.subcore_barrier()
    pltpu.sync_copy(acc.at[pl.ds(s * RPT, RPT)], p_hbm.at[c, pl.ds(s * RPT, RPT)])


def _mp_call(xws, row, col, attr, zrows):
    mesh = plsc.VectorSubcoreMesh(core_axis_name="c", subcore_axis_name="s", num_cores=NC, num_subcores=NS)
    return pl.kernel(
        _mp_body,
        out_type=jax.ShapeDtypeStruct((NC, N, D), f32),
        mesh=mesh,
        compiler_params=pltpu.CompilerParams(
            needs_layout_passes=False, use_tc_tiling_on_sc=False),
        scratch_types=[
            pltpu.VMEM((PER,), i32),
            pltpu.VMEM((NCHUNK, CE), i32),
            pltpu.VMEM((PER,), f32),
            pltpu.VMEM((CE, D), f32),
            pltpu.VMEM((CE, D), f32),
            pltpu.VMEM_SHARED((N, D), f32),
            pltpu.SemaphoreType.DMA,
            pltpu.SemaphoreType.DMA,
            pltpu.SemaphoreType.DMA,
            pltpu.SemaphoreType.DMA,
        ],
    )(xws, row, col.reshape(NW, NCHUNK, CE), attr, zrows)


def _ab_body(p0_ref, p1_ref, dis_ref, l1_ref, b1_ref, a_ref, b_ref):
    ne = (p0_ref[:] + p1_ref[:]) * dis_ref[:]
    l1 = l1_ref[:]
    dn = (((1,), (1,)), ((), ()))
    a_ref[:] = lax.dot_general(ne, l1[:, :D], dn, preferred_element_type=f32) + b1_ref[:]
    b_ref[:] = lax.dot_general(ne, l1[:, D:], dn, preferred_element_type=f32)


def _ab_call(p0, p1, dis, lin1_w, b1):
    return pl.pallas_call(
        _ab_body,
        out_shape=[jax.ShapeDtypeStruct((N, D), f32),
                   jax.ShapeDtypeStruct((N, D), f32)],
    )(p0, p1, dis, lin1_w, b1)


def _dec_body(a_hbm, b_hbm, src_hbm, trg_hbm, w2_hbm, pred_hbm,
              srcf, trgf, rowsa0, rowsa1, rowsb0, rowsb1,
              w2v, outv0, outv1, sa0, sa1, sb0, sb1):
    rowsa = [rowsa0, rowsa1]
    rowsb = [rowsb0, rowsb1]
    outv = [outv0, outv1]
    sa = [sa0, sa1]
    sb = [sb0, sb1]
    wid = lax.axis_index("s") * NC + lax.axis_index("c")
    base = wid * PER

    pltpu.sync_copy(w2_hbm, w2v)
    pltpu.sync_copy(src_hbm.at[pl.ds(base, PER)], srcf)
    pltpu.sync_copy(trg_hbm.at[pl.ds(base, PER)], trgf)
    wv = [w2v[pl.ds(j * 16, 16)] for j in range(D // 16)]
    lane = lax.iota(i32, 16)
    bperm = [jnp.bitwise_xor(lane, k) for k in (8, 4, 2, 1)]

    def stage(eoff, p):
        return (pltpu.async_copy(a_hbm.at[srcf.at[pl.ds(eoff, CE)]],
                                 rowsa[p], sa[p]),
                pltpu.async_copy(b_hbm.at[trgf.at[pl.ds(eoff, CE)]],
                                 rowsb[p], sb[p]))

    def compute(eoff, p):
        for g in range(CE // 16):
            tv = jnp.zeros((16,), f32)
            for e16 in range(16):
                e = g * 16 + e16
                ts = []
                for j in range(D // 16):
                    sl = pl.ds(j * 16, 16)
                    h = jnp.maximum(rowsa[p][e, sl] + rowsb[p][e, sl], 0.0)
                    ts.append(h * wv[j])
                while len(ts) > 1:
                    ts = [ts[k] + ts[k + 1] for k in range(0, len(ts), 2)]
                acc = ts[0]
                for pm in bperm:
                    acc = acc + acc.at[pm].get(mode="promise_in_bounds")
                tv = jnp.where(lane == e16, acc, tv)
            outv[p][pl.ds(g * 16, 16)] = tv
        pltpu.sync_copy(outv[p], pred_hbm.at[pl.ds(base + eoff, CE)])

    def wait_set(p):
        pltpu.make_async_copy(a_hbm.at[srcf.at[pl.ds(0, CE)]],
                              rowsa[p], sa[p]).wait()
        pltpu.make_async_copy(b_hbm.at[trgf.at[pl.ds(0, CE)]],
                              rowsb[p], sb[p]).wait()

    stage(0, 0)
    stage(CE, 1)

    @pl.loop(0, NCHUNK // 2)
    def _(t):
        e0 = (2 * t) * CE
        wait_set(0)
        compute(e0, 0)
        stage(e0 + 2 * CE, 0)
        wait_set(1)
        compute(e0 + CE, 1)

        @pl.when(2 * t + 3 < NCHUNK)
        def _():
            stage(e0 + 3 * CE, 1)

    if NCHUNK % 2:
        wait_set(0)
        compute((NCHUNK - 1) * CE, 0)


def _dec_call(a, b, src, trg, w2):
    mesh = plsc.VectorSubcoreMesh(core_axis_name="c", subcore_axis_name="s", num_cores=NC, num_subcores=NS)
    return pl.kernel(
        _dec_body,
        out_type=jax.ShapeDtypeStruct((E,), f32),
        mesh=mesh,
        compiler_params=pltpu.CompilerParams(
            needs_layout_passes=False, use_tc_tiling_on_sc=False),
        scratch_types=[
            pltpu.VMEM((PER,), i32),
            pltpu.VMEM((PER,), i32),
            pltpu.VMEM((CE, D), f32),
            pltpu.VMEM((CE, D), f32),
            pltpu.VMEM((CE, D), f32),
            pltpu.VMEM((CE, D), f32),
            pltpu.VMEM((D,), f32),
            pltpu.VMEM((CE,), f32),
            pltpu.VMEM((CE,), f32),
            pltpu.SemaphoreType.DMA,
            pltpu.SemaphoreType.DMA,
            pltpu.SemaphoreType.DMA,
            pltpu.SemaphoreType.DMA,
        ],
    )(a, b, src, trg, w2)


@jax.jit
def kernel(x, edge_index, edge_attr, edge_weights_index, pool_w, W0,
           w_ih, w_hh, b_ih, b_hh, lin1_w, lin1_b, lin2_w, lin2_b):
    row = edge_index[0]
    col = edge_index[1]
    src = edge_weights_index[0]
    trg = edge_weights_index[1]

    score = _score_call(x, pool_w.reshape(D, 1))
    score2d = score.reshape(125, 80)
    xw = _evolve_call(score2d, x, W0, w_ih, w_hh,
                      b_ih.reshape(1, 3 * D), b_hh.reshape(1, 3 * D))

    degp = _deg_call(col, edge_attr)
    xws, dis = _dis_call(degp, xw)

    zrows = jnp.zeros((RPT, D), f32)
    p = _mp_call(xws, row, col, edge_attr, zrows)

    a, b = _ab_call(p[0], p[1], dis, lin1_w, lin1_b.reshape(1, D))

    pred = _dec_call(a, b, src, trg, lin2_w.reshape(D))
    return pred + lin2_b[0]


# speedup vs baseline: 12.4206x; 1.0126x over previous
import jax
import jax.numpy as jnp
from jax import lax
from jax.experimental import pallas as pl
from jax.experimental.pallas import tpu as pltpu
from jax.experimental.pallas import tpu_sc as plsc

N = 10000
D = 128
E = 320000

NC = 2
NS = 16
NW = NC * NS
PER = E // NW
CE = 80
NCHUNK = PER // CE
RPT = N // NS

f32 = jnp.float32
i32 = jnp.int32


def _score_body(x_ref, pw_ref, out_ref):
    pw = pw_ref[:]
    nrm = jnp.sqrt(jnp.sum(pw * pw))
    s = jnp.dot(x_ref[:], pw, preferred_element_type=f32) / nrm
    out_ref[:] = jnp.tanh(s)


def _score_call(x, pw2):
    return pl.pallas_call(
        _score_body,
        out_shape=jax.ShapeDtypeStruct((N, 1), f32),
    )(x, pw2)


def _evolve_body(s_ref, x_ref, w0_ref, wih_ref, whh_ref, bih_ref, bhh_ref,
                 xw_ref, xt_ref):
    R, C = s_ref.shape
    iota_lin = (lax.broadcasted_iota(i32, (R, C), 0) * C
                + lax.broadcasted_iota(i32, (R, C), 1))

    def body(i, s):
        m = jnp.max(s)
        lin = jnp.min(jnp.where(s == m, iota_lin, N))
        row = x_ref[pl.ds(lin, 1), :]
        xt_ref[pl.ds(i, 1), :] = row * m
        return jnp.where(iota_lin == lin, -2.0, s)

    lax.fori_loop(0, D, body, s_ref[:])

    xt = xt_ref[:]
    w0 = w0_ref[:]
    dn = (((1,), (1,)), ((), ()))
    gi = lax.dot_general(xt, wih_ref[:], dn, preferred_element_type=f32) + bih_ref[:]
    gh = lax.dot_general(w0, whh_ref[:], dn, preferred_element_type=f32) + bhh_ref[:]
    i_r, i_z, i_n = gi[:, :D], gi[:, D:2 * D], gi[:, 2 * D:]
    h_r, h_z, h_n = gh[:, :D], gh[:, D:2 * D], gh[:, 2 * D:]
    r = jax.nn.sigmoid(i_r + h_r)
    z = jax.nn.sigmoid(i_z + h_z)
    ncand = jnp.tanh(i_n + r * h_n)
    w = (1.0 - z) * ncand + z * w0
    xw_ref[:] = jnp.dot(x_ref[:], w, preferred_element_type=f32)


def _evolve_call(score2d, x, w0, w_ih, w_hh, b_ih2, b_hh2):
    return pl.pallas_call(
        _evolve_body,
        out_shape=jax.ShapeDtypeStruct((N, D), f32),
        scratch_shapes=[pltpu.VMEM((D, D), f32)],
    )(score2d, x, w0, w_ih, w_hh, b_ih2, b_hh2)


def _deg_body(col_hbm, attr_hbm, degp_hbm, colv, attrv, deg, sem):
    wid = lax.axis_index("s") * NC + lax.axis_index("c")
    base = wid * PER

    @pl.loop(0, N // 16, unroll=8)
    def _(j):
        deg[pl.ds(j * 16, 16)] = jnp.zeros((16,), f32)

    pltpu.sync_copy(col_hbm.at[pl.ds(base, PER)], colv)
    pltpu.sync_copy(attr_hbm.at[pl.ds(base, PER)], attrv)

    @pl.loop(0, PER // 16, unroll=8)
    def _(j):
        idx = colv[pl.ds(j * 16, 16)]
        a = attrv[pl.ds(j * 16, 16)]
        plsc.addupdate_scatter(deg, [idx], a)

    pltpu.sync_copy(deg, degp_hbm.at[wid])


def _deg_call(col, attr):
    mesh = plsc.VectorSubcoreMesh(core_axis_name="c", subcore_axis_name="s", num_cores=NC, num_subcores=NS)
    return pl.kernel(
        _deg_body,
        out_type=jax.ShapeDtypeStruct((NW, N), f32),
        mesh=mesh,
        compiler_params=pltpu.CompilerParams(
            needs_layout_passes=False, use_tc_tiling_on_sc=False),
        scratch_types=[
            pltpu.VMEM((PER,), i32),
            pltpu.VMEM((PER,), f32),
            pltpu.VMEM((N,), f32),
            pltpu.SemaphoreType.DMA,
        ],
    )(col, attr)


def _dis_body(degp_ref, xw_ref, xws_ref, dis_ref):
    deg = jnp.sum(degp_ref[:], axis=0)
    dis = jnp.where(deg > 0, lax.rsqrt(jnp.maximum(deg, 1e-12)), 0.0)
    disc = dis[:, None]
    dis_ref[:] = disc
    xws_ref[:] = xw_ref[:] * disc


def _dis_call(degp, xw):
    return pl.pallas_call(
        _dis_body,
        out_shape=[jax.ShapeDtypeStruct((N, D), f32),
                   jax.ShapeDtypeStruct((N, 1), f32)],
    )(degp, xw)


def _mp_body(xws_hbm, row_hbm, col3_hbm, attr_hbm, zrows_hbm, p_hbm,
             rowf, colf2, attrf, rows0, rows1,
             acc, gsem0, gsem1, ssem0, ssem1):
    rows = [rows0, rows1]
    gsem = [gsem0, gsem1]
    ssem = [ssem0, ssem1]
    c = lax.axis_index("c")
    s = lax.axis_index("s")
    wid = s * NC + c
    base = wid * PER

    pltpu.sync_copy(zrows_hbm, acc.at[pl.ds(s * RPT, RPT)])
    pltpu.sync_copy(row_hbm.at[pl.ds(base, PER)], rowf)
    pltpu.sync_copy(col3_hbm.at[wid], colf2)
    pltpu.sync_copy(attr_hbm.at[pl.ds(base, PER)], attrf)
    plsc.subcore_barrier()

    def scale(p, ebase):
        for g in range(CE // 16):
            a16 = attrf[pl.ds(ebase + g * 16, 16)]
            for e16 in range(16):
                e = g * 16 + e16
                av = a16.at[jnp.full((16,), e16, i32)].get(
                    mode="promise_in_bounds")
                for j in range(D // 16):
                    sl = pl.ds(j * 16, 16)
                    rows[p][e, sl] = rows[p][e, sl] * av

    def stage(eoff, p):
        return pltpu.async_copy(
            xws_hbm.at[rowf.at[pl.ds(eoff, CE)]], rows[p], gsem[p])

    def wait_gather(p):
        pltpu.make_async_copy(xws_hbm.at[rowf.at[pl.ds(0, CE)]],
                              rows[p], gsem[p]).wait()

    stage(0, 0)
    stage(CE, 1)

    @pl.loop(0, NCHUNK // 2)
    def _(t):
        e0 = (2 * t) * CE
        wait_gather(0)
        scale(0, e0)
        s0 = pltpu.async_copy(rows[0], acc.at[colf2.at[2 * t]], ssem[0],
                              add=True)
        wait_gather(1)
        scale(1, e0 + CE)
        s1 = pltpu.async_copy(rows[1], acc.at[colf2.at[2 * t + 1]], ssem[1],
                              add=True)
        s0.wait()
        stage(e0 + 2 * CE, 0)
        s1.wait()

        @pl.when(2 * t + 3 < NCHUNK)
        def _():
            stage(e0 + 3 * CE, 1)

    if NCHUNK % 2:
        wait_gather(0)
        scale(0, (NCHUNK - 1) * CE)
        pltpu.async_copy(rows[0], acc.at[colf2.at[NCHUNK - 1]], ssem[0],
                         add=True).wait()

    plsc.subcore_barrier()
    pltpu.sync_copy(acc.at[pl.ds(s * RPT, RPT)], p_hbm.at[c, pl.ds(s * RPT, RPT)])


def _mp_call(xws, row, col, attr, zrows):
    mesh = plsc.VectorSubcoreMesh(core_axis_name="c", subcore_axis_name="s", num_cores=NC, num_subcores=NS)
    return pl.kernel(
        _mp_body,
        out_type=jax.ShapeDtypeStruct((NC, N, D), f32),
        mesh=mesh,
        compiler_params=pltpu.CompilerParams(
            needs_layout_passes=False, use_tc_tiling_on_sc=False),
        scratch_types=[
            pltpu.VMEM((PER,), i32),
            pltpu.VMEM((NCHUNK, CE), i32),
            pltpu.VMEM((PER,), f32),
            pltpu.VMEM((CE, D), f32),
            pltpu.VMEM((CE, D), f32),
            pltpu.VMEM_SHARED((N, D), f32),
            pltpu.SemaphoreType.DMA,
            pltpu.SemaphoreType.DMA,
            pltpu.SemaphoreType.DMA,
            pltpu.SemaphoreType.DMA,
        ],
    )(xws, row, col.reshape(NW, NCHUNK, CE), attr, zrows)


def _ab_body(p0_ref, p1_ref, dis_ref, l1_ref, b1_ref, a_ref, b_ref):
    ne = (p0_ref[:] + p1_ref[:]) * dis_ref[:]
    l1 = l1_ref[:]
    dn = (((1,), (1,)), ((), ()))
    a_ref[:] = lax.dot_general(ne, l1[:, :D], dn, preferred_element_type=f32) + b1_ref[:]
    b_ref[:] = lax.dot_general(ne, l1[:, D:], dn, preferred_element_type=f32)


def _ab_call(p0, p1, dis, lin1_w, b1):
    return pl.pallas_call(
        _ab_body,
        out_shape=[jax.ShapeDtypeStruct((N, D), f32),
                   jax.ShapeDtypeStruct((N, D), f32)],
    )(p0, p1, dis, lin1_w, b1)


def _dec_body(a_hbm, b_hbm, src_hbm, trg_hbm, w2_hbm, pred_hbm,
              srcf, trgf, rowsa0, rowsa1, rowsb0, rowsb1,
              w2v, outv0, outv1, sa0, sa1, sb0, sb1):
    rowsa = [rowsa0, rowsa1]
    rowsb = [rowsb0, rowsb1]
    outv = [outv0, outv1]
    sa = [sa0, sa1]
    sb = [sb0, sb1]
    wid = lax.axis_index("s") * NC + lax.axis_index("c")
    base = wid * PER

    pltpu.sync_copy(w2_hbm, w2v)
    pltpu.sync_copy(src_hbm.at[pl.ds(base, PER)], srcf)
    pltpu.sync_copy(trg_hbm.at[pl.ds(base, PER)], trgf)
    wv = [w2v[pl.ds(j * 16, 16)] for j in range(D // 16)]
    lane = lax.iota(i32, 16)
    bperm = [jnp.bitwise_xor(lane, k) for k in (8, 4, 2, 1)]

    def stage(eoff, p):
        return (pltpu.async_copy(a_hbm.at[srcf.at[pl.ds(eoff, CE)]],
                                 rowsa[p], sa[p]),
                pltpu.async_copy(b_hbm.at[trgf.at[pl.ds(eoff, CE)]],
                                 rowsb[p], sb[p]))

    def compute(eoff, p):
        for g in range(CE // 16):
            tv = jnp.zeros((16,), f32)
            for e16 in range(16):
                e = g * 16 + e16
                ts = []
                for j in range(D // 16):
                    sl = pl.ds(j * 16, 16)
                    h = jnp.maximum(rowsa[p][e, sl] + rowsb[p][e, sl], 0.0)
                    ts.append(h * wv[j])
                while len(ts) > 1:
                    ts = [ts[k] + ts[k + 1] for k in range(0, len(ts), 2)]
                acc = ts[0]
                for pm in bperm:
                    acc = acc + acc.at[pm].get(mode="promise_in_bounds")
                tv = jnp.where(lane == e16, acc, tv)
            outv[p][pl.ds(g * 16, 16)] = tv
        pltpu.sync_copy(outv[p], pred_hbm.at[pl.ds(base + eoff, CE)])

    def wait_set(p):
        pltpu.make_async_copy(a_hbm.at[srcf.at[pl.ds(0, CE)]],
                              rowsa[p], sa[p]).wait()
        pltpu.make_async_copy(b_hbm.at[trgf.at[pl.ds(0, CE)]],
                              rowsb[p], sb[p]).wait()

    stage(0, 0)
    stage(CE, 1)

    @pl.loop(0, NCHUNK // 2)
    def _(t):
        e0 = (2 * t) * CE
        wait_set(0)
        compute(e0, 0)
        stage(e0 + 2 * CE, 0)
        wait_set(1)
        compute(e0 + CE, 1)

        @pl.when(2 * t + 3 < NCHUNK)
        def _():
            stage(e0 + 3 * CE, 1)

    if NCHUNK % 2:
        wait_set(0)
        compute((NCHUNK - 1) * CE, 0)


def _dec_call(a, b, src, trg, w2):
    mesh = plsc.VectorSubcoreMesh(core_axis_name="c", subcore_axis_name="s", num_cores=NC, num_subcores=NS)
    return pl.kernel(
        _dec_body,
        out_type=jax.ShapeDtypeStruct((E,), f32),
        mesh=mesh,
        compiler_params=pltpu.CompilerParams(
            needs_layout_passes=False, use_tc_tiling_on_sc=False),
        scratch_types=[
            pltpu.VMEM((PER,), i32),
            pltpu.VMEM((PER,), i32),
            pltpu.VMEM((CE, D), f32),
            pltpu.VMEM((CE, D), f32),
            pltpu.VMEM((CE, D), f32),
            pltpu.VMEM((CE, D), f32),
            pltpu.VMEM((D,), f32),
            pltpu.VMEM((CE,), f32),
            pltpu.VMEM((CE,), f32),
            pltpu.SemaphoreType.DMA,
            pltpu.SemaphoreType.DMA,
            pltpu.SemaphoreType.DMA,
            pltpu.SemaphoreType.DMA,
        ],
    )(a, b, src, trg, w2)


@jax.jit
def kernel(x, edge_index, edge_attr, edge_weights_index, pool_w, W0,
           w_ih, w_hh, b_ih, b_hh, lin1_w, lin1_b, lin2_w, lin2_b):
    row = edge_index[0]
    col = edge_index[1]
    src = edge_weights_index[0]
    trg = edge_weights_index[1]

    score = _score_call(x, pool_w.reshape(D, 1))
    score2d = score.reshape(125, 80)
    xw = _evolve_call(score2d, x, W0, w_ih, w_hh,
                      b_ih.reshape(1, 3 * D), b_hh.reshape(1, 3 * D))

    degp = _deg_call(col, edge_attr)
    xws, dis = _dis_call(degp, xw)

    zrows = jnp.zeros((RPT, D), f32)
    p = _mp_call(xws, row, col, edge_attr, zrows)

    a, b = _ab_call(p[0], p[1], dis, lin1_w, lin1_b.reshape(1, D))

    pred = _dec_call(a, b, src, trg, lin2_w.reshape(D))
    return pred + lin2_b[0]
